# baseline (speedup 1.0000x reference)
import functools

import jax
import jax.numpy as jnp
from jax import lax
from jax.experimental import pallas as pl
from jax.experimental.pallas import tpu as pltpu
from jax.experimental.pallas import tpu_sc as plsc

VOCAB = 50257
SEQ = 2048
HID = 768
BATCH = 4

NUM_CORES = 2
NUM_SUBCORES = 16
NW = NUM_CORES * NUM_SUBCORES
S_PER_W = SEQ // NW
LANES = 16
VECS_PER_ROW = HID // LANES


def _build():
    mesh = plsc.VectorSubcoreMesh(core_axis_name="c", subcore_axis_name="s")

    @functools.partial(
        pl.kernel,
        mesh=mesh,
        out_type=jax.ShapeDtypeStruct((BATCH * SEQ, HID), jnp.float32),
        scratch_types=[
            pltpu.VMEM((S_PER_W,), jnp.int32),
            pltpu.VMEM((S_PER_W, HID), jnp.float32),
            pltpu.VMEM((S_PER_W, HID), jnp.float32),
            pltpu.SemaphoreType.DMA,
        ],
    )
    def embed(ids_hbm, table_hbm, pos_hbm, out_hbm, idx_v, rows_v, pos_v, sem):
        wid = lax.axis_index("s") * NUM_CORES + lax.axis_index("c")
        s_base = wid * S_PER_W
        pltpu.sync_copy(pos_hbm.at[pl.ds(s_base, S_PER_W)], pos_v)
        for b in range(BATCH):
            flat = b * SEQ + s_base
            pltpu.sync_copy(ids_hbm.at[pl.ds(flat, S_PER_W)], idx_v)
            pltpu.async_copy(table_hbm.at[idx_v], rows_v, sem).wait()

            def body(r, carry):
                for c in range(VECS_PER_ROW):
                    sl = pl.ds(c * LANES, LANES)
                    rows_v[r, sl] = rows_v[r, sl] + pos_v[r, sl]
                return carry

            lax.fori_loop(0, S_PER_W, body, 0)
            pltpu.sync_copy(rows_v, out_hbm.at[pl.ds(flat, S_PER_W)])

    return embed


_embed = _build()


def kernel(input_ids, token_embeddings, position_embeddings):
    ids_flat = input_ids.reshape(-1).astype(jnp.int32)
    out = _embed(ids_flat, token_embeddings, position_embeddings)
    return out.reshape(BATCH, SEQ, HID)
